# Initial kernel scaffold; baseline (speedup 1.0000x reference)
import jax
import jax.numpy as jnp
from jax.experimental import pallas as pl


def kernel(input_ids, W, P, gamma, beta):
    raise NotImplementedError("write your pallas kernel here")


# speedup vs baseline: 1.5512x; 1.5512x over previous
import functools

import jax
import jax.numpy as jnp
from jax import lax
from jax.experimental import pallas as pl
from jax.experimental.pallas import tpu as pltpu
from jax.experimental.pallas import tpu_sc as plsc

HID = 1024
EPS = 1e-5
POS_OFF = 2
NC, NS = 2, 16
NW = NC * NS
CHUNK = 32


def _gather_call(ids3, table, n_rows, nch):
    mesh = plsc.VectorSubcoreMesh(core_axis_name="c", subcore_axis_name="s")

    @functools.partial(
        pl.kernel,
        mesh=mesh,
        out_type=jax.ShapeDtypeStruct((n_rows, HID), jnp.float32),
        scratch_types=[
            pltpu.VMEM((nch, CHUNK), jnp.int32),
            pltpu.VMEM((CHUNK, HID), jnp.float32),
            pltpu.VMEM((CHUNK, HID), jnp.float32),
            pltpu.SemaphoreType.DMA,
            pltpu.SemaphoreType.DMA,
        ],
    )
    def k(ids_hbm, table_hbm, out_hbm, idx_v, buf0, buf1, sem0, sem1):
        wid = lax.axis_index("s") * NC + lax.axis_index("c")
        base = wid * (nch * CHUNK)
        pltpu.sync_copy(ids_hbm.at[wid], idx_v)
        bufs = (buf0, buf1)
        sems = (sem0, sem1)
        copies = [None, None]
        copies[0] = pltpu.async_copy(table_hbm.at[idx_v.at[0]], bufs[0], sems[0])
        for c in range(nch):
            cur = c & 1
            nxt = (c + 1) & 1
            if c + 1 < nch:
                copies[nxt] = pltpu.async_copy(
                    table_hbm.at[idx_v.at[c + 1]], bufs[nxt], sems[nxt])
            copies[cur].wait()
            pltpu.sync_copy(bufs[cur], out_hbm.at[pl.ds(base + c * CHUNK, CHUNK)])

    return k(ids3, table)


def _ln_body(x_ref, p_ref, g_ref, b_ref, o_ref):
    x = x_ref[0] + p_ref[...]
    mean = jnp.mean(x, axis=-1, keepdims=True)
    xc = x - mean
    var = jnp.mean(xc * xc, axis=-1, keepdims=True)
    o_ref[0] = xc * lax.rsqrt(var + EPS) * g_ref[...] + b_ref[...]


def kernel(input_ids, W, P, gamma, beta):
    B, S = input_ids.shape
    n = B * S
    nch = n // (NW * CHUNK)
    ids3 = input_ids.astype(jnp.int32).reshape(NW, nch, CHUNK)
    G = _gather_call(ids3, W, n, nch).reshape(B, S, HID)
    p_slice = lax.slice(P, (POS_OFF, 0), (POS_OFF + S, HID))
    bs = 256
    out = pl.pallas_call(
        _ln_body,
        grid=(B, S // bs),
        in_specs=[
            pl.BlockSpec((1, bs, HID), lambda b, s: (b, s, 0)),
            pl.BlockSpec((bs, HID), lambda b, s: (s, 0)),
            pl.BlockSpec((1, HID), lambda b, s: (0, 0)),
            pl.BlockSpec((1, HID), lambda b, s: (0, 0)),
        ],
        out_specs=pl.BlockSpec((1, bs, HID), lambda b, s: (b, s, 0)),
        out_shape=jax.ShapeDtypeStruct((B, S, HID), jnp.float32),
    )(G, p_slice, gamma.reshape(1, HID), beta.reshape(1, HID))
    return out


# speedup vs baseline: 1.6928x; 1.0913x over previous
import functools

import jax
import jax.numpy as jnp
from jax import lax
from jax.experimental import pallas as pl
from jax.experimental.pallas import tpu as pltpu
from jax.experimental.pallas import tpu_sc as plsc

HID = 1024
EPS = 1e-5
POS_OFF = 2
NC, NS = 2, 16
NW = NC * NS
CHUNK = 32


def _gather_call(ids3, table, n_rows, nch):
    mesh = plsc.VectorSubcoreMesh(core_axis_name="c", subcore_axis_name="s")

    @functools.partial(
        pl.kernel,
        mesh=mesh,
        out_type=jax.ShapeDtypeStruct((n_rows, HID), jnp.float32),
        scratch_types=[
            pltpu.VMEM((nch, CHUNK), jnp.int32),
            pltpu.VMEM((CHUNK, HID), jnp.float32),
            pltpu.VMEM((CHUNK, HID), jnp.float32),
            pltpu.SemaphoreType.DMA,
            pltpu.SemaphoreType.DMA,
        ],
    )
    def k(ids_hbm, table_hbm, out_hbm, idx_v, buf0, buf1, sem0, sem1):
        wid = lax.axis_index("s") * NC + lax.axis_index("c")
        base = wid * (nch * CHUNK)
        pltpu.sync_copy(ids_hbm.at[wid], idx_v)
        bufs = (buf0, buf1)
        sems = (sem0, sem1)
        copies = [None, None]
        copies[0] = pltpu.async_copy(table_hbm.at[idx_v.at[0]], bufs[0], sems[0])
        for c in range(nch):
            cur = c & 1
            nxt = (c + 1) & 1
            if c + 1 < nch:
                copies[nxt] = pltpu.async_copy(
                    table_hbm.at[idx_v.at[c + 1]], bufs[nxt], sems[nxt])
            copies[cur].wait()
            pltpu.sync_copy(bufs[cur], out_hbm.at[pl.ds(base + c * CHUNK, CHUNK)])

    return k(ids3, table)


def _ln_body(x_ref, p_ref, g_ref, b_ref, o_ref):
    x = x_ref[...] + p_ref[...]
    mean = jnp.mean(x, axis=-1, keepdims=True)
    xc = x - mean
    var = jnp.mean(xc * xc, axis=-1, keepdims=True)
    o_ref[...] = xc * lax.rsqrt(var + EPS) * g_ref[...] + b_ref[...]


def kernel(input_ids, W, P, gamma, beta):
    B, S = input_ids.shape
    n = B * S
    nch = n // (NW * CHUNK)
    ids3 = input_ids.astype(jnp.int32).reshape(NW, nch, CHUNK)
    G = _gather_call(ids3, W, n, nch)
    p_slice = lax.slice(P, (POS_OFF, 0), (POS_OFF + S, HID))
    bs = 512
    nsb = S // bs
    out = pl.pallas_call(
        _ln_body,
        grid=(n // bs,),
        in_specs=[
            pl.BlockSpec((bs, HID), lambda i: (i, 0)),
            pl.BlockSpec((bs, HID), lambda i: (i % nsb, 0)),
            pl.BlockSpec((1, HID), lambda i: (0, 0)),
            pl.BlockSpec((1, HID), lambda i: (0, 0)),
        ],
        out_specs=pl.BlockSpec((bs, HID), lambda i: (i, 0)),
        out_shape=jax.ShapeDtypeStruct((n, HID), jnp.float32),
    )(G, p_slice, gamma.reshape(1, HID), beta.reshape(1, HID))
    return out.reshape(B, S, HID)


# speedup vs baseline: 1.7555x; 1.0370x over previous
import functools

import jax
import jax.numpy as jnp
from jax import lax
from jax.experimental import pallas as pl
from jax.experimental.pallas import tpu as pltpu
from jax.experimental.pallas import tpu_sc as plsc

HID = 1024
EPS = 1e-5
POS_OFF = 2
NC, NS = 2, 16
NW = NC * NS
CHUNK = 32


def _gather_call(ids3, table, n_rows, nch):
    mesh = plsc.VectorSubcoreMesh(core_axis_name="c", subcore_axis_name="s")

    @functools.partial(
        pl.kernel,
        mesh=mesh,
        out_type=jax.ShapeDtypeStruct((n_rows, HID), jnp.float32),
        scratch_types=[
            pltpu.VMEM((nch, CHUNK), jnp.int32),
            pltpu.VMEM((CHUNK, HID), jnp.float32),
            pltpu.VMEM((CHUNK, HID), jnp.float32),
            pltpu.SemaphoreType.DMA,
            pltpu.SemaphoreType.DMA,
        ],
    )
    def k(ids_hbm, table_hbm, out_hbm, idx_v, buf0, buf1, sem0, sem1):
        wid = lax.axis_index("s") * NC + lax.axis_index("c")
        base = wid * (nch * CHUNK)
        pltpu.sync_copy(ids_hbm.at[wid], idx_v)
        bufs = (buf0, buf1)
        sems = (sem0, sem1)
        copies = [None, None]
        copies[0] = pltpu.async_copy(table_hbm.at[idx_v.at[0]], bufs[0], sems[0])
        for c in range(nch):
            cur = c & 1
            nxt = (c + 1) & 1
            if c + 1 < nch:
                copies[nxt] = pltpu.async_copy(
                    table_hbm.at[idx_v.at[c + 1]], bufs[nxt], sems[nxt])
            copies[cur].wait()
            pltpu.sync_copy(bufs[cur], out_hbm.at[pl.ds(base + c * CHUNK, CHUNK)])

    return k(ids3, table)


def _ln_body(x_ref, p_ref, g_ref, b_ref, o_ref):
    x = x_ref[...] + p_ref[...]
    mean = jnp.mean(x, axis=-1, keepdims=True)
    xc = x - mean
    var = jnp.mean(xc * xc, axis=-1, keepdims=True)
    o_ref[...] = xc * lax.rsqrt(var + EPS) * g_ref[...] + b_ref[...]


def kernel(input_ids, W, P, gamma, beta):
    B, S = input_ids.shape
    n = B * S
    nch = n // (NW * CHUNK)
    ids3 = input_ids.astype(jnp.int32).reshape(NW, nch, CHUNK)
    G = _gather_call(ids3, W, n, nch)
    p_slice = lax.slice(P, (POS_OFF, 0), (POS_OFF + S, HID))
    bs = 512
    nsb = S // bs
    out = pl.pallas_call(
        _ln_body,
        grid=(nsb, B),
        in_specs=[
            pl.BlockSpec((bs, HID), lambda s, b: (b * nsb + s, 0)),
            pl.BlockSpec((bs, HID), lambda s, b: (s, 0)),
            pl.BlockSpec((1, HID), lambda s, b: (0, 0)),
            pl.BlockSpec((1, HID), lambda s, b: (0, 0)),
        ],
        out_specs=pl.BlockSpec((bs, HID), lambda s, b: (b * nsb + s, 0)),
        out_shape=jax.ShapeDtypeStruct((n, HID), jnp.float32),
    )(G, p_slice, gamma.reshape(1, HID), beta.reshape(1, HID))
    return out.reshape(B, S, HID)
